# Initial kernel scaffold; baseline (speedup 1.0000x reference)
import jax
import jax.numpy as jnp
from jax.experimental import pallas as pl


def kernel(x, edge_index, edge_attr, W, root, bias, conv_w, conv_b):
    raise NotImplementedError("write your pallas kernel here")


# speedup vs baseline: 29.3164x; 29.3164x over previous
import functools
import jax
import jax.numpy as jnp
from jax import lax
from jax.experimental import pallas as pl
from jax.experimental.pallas import tpu as pltpu
from jax.experimental.pallas import tpu_sc as plsc

_KD = 5
_NB = 4
_NQ = 8


def _elu(v):
    return jnp.where(v > 0, v, jnp.exp(v) - 1.0)


def _bd8(m):
    eye = jnp.eye(8, dtype=jnp.float32)
    return (eye[:, None, :, None] * m.astype(jnp.float32)[None, :, None, :]).reshape(128, 128)


def _prep_body(E, V, r_ref, c_ref, a0_ref, a1_ref, gb_ref, rm_ref, wg_ref):
    r = r_ref[...]
    c = c_ref[...]
    a0 = a0_ref[...]
    a1 = a1_ref[...]
    v0 = a0 * (_KD - 1.0)
    b0 = jnp.floor(v0)
    f0 = v0 - b0
    i0 = b0.astype(jnp.int32)
    v1 = a1 * (_KD - 1.0)
    b1 = jnp.floor(v1)
    f1 = v1 - b1
    i1 = b1.astype(jnp.int32)
    valid = lax.broadcasted_iota(jnp.int32, r.shape, 1) < E
    for s in range(4):
        bit0 = s & 1
        bit1 = (s >> 1) & 1
        basis = (f0 if bit0 else 1.0 - f0) * (f1 if bit1 else 1.0 - f1)
        wi = jnp.clip(i0 + bit0, 0, _KD - 1) + _KD * jnp.clip(i1 + bit1, 0, _KD - 1)
        gb_ref[s : s + 1, :] = (wi * V + c) * _NB
        rm_ref[s : s + 1, :] = r
        wg_ref[s : s + 1, :] = jnp.where(valid, basis, 0.0)


def _y_body(x_ref, w_ref, y_ref):
    y_ref[...] = jnp.dot(x_ref[...], w_ref[0], preferred_element_type=jnp.float32)


def _comb_body(x_ref, agg_ref, root_ref, cw_ref, b_ref, cb_ref, o_ref):
    xb = x_ref[...]
    agg = agg_ref[0] + agg_ref[1]
    h = agg + jnp.dot(xb, root_ref[...], preferred_element_type=jnp.float32) + b_ref[...]
    h = _elu(h)
    res = _elu(jnp.dot(xb, cw_ref[...], preferred_element_type=jnp.float32) + cb_ref[...])
    o_ref[...] = _elu(h + res)


def _make_sc_agg(V, C, PAIRS):
    SCH = 1024
    CH = 256
    SLICE = PAIRS // _NQ
    NSC = SLICE // SCH
    ROWS_SH = _NB * V
    mesh = plsc.VectorSubcoreMesh(core_axis_name="c", subcore_axis_name="s")
    NC, NS = mesh.num_cores, mesh.num_subcores
    ZT = 10
    ZR = ROWS_SH // ZT

    @functools.partial(
        pl.kernel,
        out_type=jax.ShapeDtypeStruct((NC, ROWS_SH, 8 * C), jnp.float32),
        mesh=mesh,
        scratch_types=[
            pltpu.VMEM_SHARED((ROWS_SH, 8 * C), jnp.float32),
            pltpu.VMEM((8, 128), jnp.int32),
            pltpu.VMEM((8, 128), jnp.int32),
            pltpu.VMEM((SCH,), jnp.float32),
            pltpu.VMEM((CH, 8 * C), jnp.float32),
            pltpu.SemaphoreType.DMA,
        ],
    )
    def sc_agg(y5, gb2, rm2, wg, out_hbm, acc_sh, gbv, rmv, wgv, rows, sem):
        cid = lax.axis_index("c")
        sid = lax.axis_index("s")
        b = sid % _NB
        q = cid * (_NQ // NC) + sid // _NB
        bv = b * V

        zero16 = jnp.zeros((16,), jnp.float32)

        def zb(i, carry):
            for j8 in range(8):
                rows[i, pl.ds(j8 * 16, 16)] = zero16
            return carry

        lax.fori_loop(0, CH, zb, 0)

        @pl.when(sid < ZT)
        def _():
            z0 = sid * ZR
            done = 0
            while done < ZR:
                n = min(CH, ZR - done)
                pltpu.sync_copy(rows.at[pl.ds(0, n)], acc_sh.at[pl.ds(z0 + done, n)])
                done += n

        plsc.subcore_barrier()

        prow = q * (SLICE // 128)
        ppair = q * SLICE

        def cb(ci, carry):
            ro = prow + ci * 8
            pltpu.sync_copy(gb2.at[pl.ds(ro, 8)], gbv)
            pltpu.sync_copy(rm2.at[pl.ds(ro, 8)], rmv)
            pltpu.sync_copy(wg.at[pl.ds(ppair + ci * SCH, SCH)], wgv)
            for j in range(8):
                for l in range(8):
                    sl = pl.ds(l * 16, 16)
                    gbv[j, sl] = gbv[j, sl] + b
                    rmv[j, sl] = rmv[j, sl] + bv
            for g4 in range(SCH // CH):
                cps = [
                    pltpu.async_copy(
                        y5.at[gbv.at[g4 * 2 + j]], rows.at[pl.ds(j * 128, 128)], sem
                    )
                    for j in range(2)
                ]
                for cp in cps:
                    cp.wait()

                def pb(t, carry2):
                    base = t * 16
                    wg16 = wgv[pl.ds(g4 * CH + base, 16)]
                    for i in range(16):
                        w = wg16[i]
                        p = base + i
                        for j8 in range(8):
                            sl = pl.ds(j8 * 16, 16)
                            rows[p, sl] = rows[p, sl] * w
                    return carry2

                lax.fori_loop(0, CH // 16, pb, 0)
                for j in range(2):
                    pltpu.sync_copy(
                        rows.at[pl.ds(j * 128, 128)],
                        acc_sh.at[rmv.at[g4 * 2 + j]],
                        add=True,
                    )
            return carry

        lax.fori_loop(0, NSC, cb, 0)
        plsc.subcore_barrier()

        @pl.when(sid < ZT)
        def _():
            pltpu.sync_copy(
                acc_sh.at[pl.ds(sid * ZR, ZR)],
                out_hbm.at[cid, pl.ds(sid * ZR, ZR)],
            )

    return sc_agg


def kernel(x, edge_index, edge_attr, W, root, bias, conv_w, conv_b):
    N, V, C, T = x.shape
    G = N * T
    E = edge_index.shape[1] // N
    K = W.shape[0]
    EP = ((E + 2047) // 2048) * 2048
    PAIRS = 4 * EP

    x = x.astype(jnp.float32)
    xt = jnp.transpose(x, (1, 3, 0, 2)).reshape(V, G, C)
    X5 = xt.reshape(V, _NB, 8 * C).reshape(V * _NB, 8 * C)
    Xblk = jnp.transpose(xt.reshape(V, _NB, 8 * C), (1, 0, 2)).reshape(_NB * V, 8 * C)

    r = edge_index[0, :E].astype(jnp.int32)
    c = edge_index[1, :E].astype(jnp.int32)
    rp = jnp.pad(r, (0, EP - E)).reshape(1, EP)
    cp_ = jnp.pad(c, (0, EP - E)).reshape(1, EP)
    a0 = jnp.pad(edge_attr[:E, 0].astype(jnp.float32), (0, EP - E)).reshape(1, EP)
    a1 = jnp.pad(edge_attr[:E, 1].astype(jnp.float32), (0, EP - E)).reshape(1, EP)

    gb, rm, wg = pl.pallas_call(
        functools.partial(_prep_body, E, V),
        out_shape=[
            jax.ShapeDtypeStruct((4, EP), jnp.int32),
            jax.ShapeDtypeStruct((4, EP), jnp.int32),
            jax.ShapeDtypeStruct((4, EP), jnp.float32),
        ],
    )(rp, cp_, a0, a1)

    gb2 = gb.reshape(PAIRS // 128, 128)
    rm2 = rm.reshape(PAIRS // 128, 128)
    wgf = wg.reshape(PAIRS)

    BDW = jnp.stack([_bd8(W[k]) for k in range(K)])
    Y5 = pl.pallas_call(
        _y_body,
        grid=(K,),
        in_specs=[
            pl.BlockSpec((V * _NB, 8 * C), lambda k: (0, 0)),
            pl.BlockSpec((1, 8 * C, 8 * C), lambda k: (k, 0, 0)),
        ],
        out_specs=pl.BlockSpec((V * _NB, 8 * C), lambda k: (k, 0)),
        out_shape=jax.ShapeDtypeStruct((K * V * _NB, 8 * C), jnp.float32),
    )(X5, BDW)

    agg = _make_sc_agg(V, C, PAIRS)(Y5, gb2, rm2, wgf)

    BDroot = _bd8(root)
    BDconv = _bd8(jnp.transpose(conv_w))
    biasb = jnp.tile(bias.astype(jnp.float32), 8).reshape(1, 8 * C)
    convbb = jnp.tile(conv_b.astype(jnp.float32), 8).reshape(1, 8 * C)

    MB2 = 2000
    Yblk = pl.pallas_call(
        _comb_body,
        grid=((_NB * V) // MB2,),
        in_specs=[
            pl.BlockSpec((MB2, 8 * C), lambda m: (m, 0)),
            pl.BlockSpec((2, MB2, 8 * C), lambda m: (0, m, 0)),
            pl.BlockSpec((8 * C, 8 * C), lambda m: (0, 0)),
            pl.BlockSpec((8 * C, 8 * C), lambda m: (0, 0)),
            pl.BlockSpec((1, 8 * C), lambda m: (0, 0)),
            pl.BlockSpec((1, 8 * C), lambda m: (0, 0)),
        ],
        out_specs=pl.BlockSpec((MB2, 8 * C), lambda m: (m, 0)),
        out_shape=jax.ShapeDtypeStruct((_NB * V, 8 * C), jnp.float32),
    )(Xblk, agg, BDroot, BDconv, biasb, convbb)

    out = Yblk.reshape(_NB, V, 8, C)
    out = jnp.transpose(out, (1, 0, 2, 3)).reshape(V, T, N, C)
    return jnp.transpose(out, (2, 0, 3, 1))
